# baseline (speedup 1.0000x reference)
import jax
import jax.numpy as jnp
from jax.experimental import pallas as pl
from jax.experimental.pallas import tpu as pltpu
from jax.experimental.pallas import tpu_sc as plsc

NUM_CODES = 8192
DIM = 256
TM = 512
TN = 2048
NT = 8192 // TM
NC = NUM_CODES // TN
LOSS_SCALE = 0.25 / 2097152.0


def _row_sumsq(a):
    aa = a * a
    return jnp.sum(aa[:, :128] + aa[:, 128:], axis=1)


def _fold_min(vals):
    rows = vals.shape[0]
    while rows > 1:
        half = rows // 2
        vals = jnp.minimum(vals[:half], vals[half:])
        rows = half
    return vals[0]


def _vq_tc_body(x_ref, w_ref, idx_ref, loss_ref, w16_ref, sw_ref):
    tb = pl.program_id(0)

    @pl.when(tb == 0)
    def _init():
        w_full = w_ref[...]
        w16_ref[...] = w_full.astype(jnp.bfloat16)
        sw_ref[...] = _row_sumsq(w_full)[:, None]

    x = x_ref[...]
    sx = _row_sumsq(x)
    x16 = x.astype(jnp.bfloat16)
    iota = jax.lax.broadcasted_iota(jnp.int32, (TN, TM), 0)

    best_val = None
    best_idx = None
    for ci in range(NC):
        w16 = w16_ref[ci * TN:(ci + 1) * TN, :]
        sw = sw_ref[ci * TN:(ci + 1) * TN, :]
        mm = jax.lax.dot_general(
            w16, x16,
            (((1,), (1,)), ((), ())),
            preferred_element_type=jnp.float32)
        d = (sw + sx[None, :]) - 2.0 * mm
        lval = _fold_min(d)
        lidx = _fold_min(jnp.where(d == lval[None, :], iota, 2 ** 30))
        lidx = lidx + ci * TN
        if ci == 0:
            best_val, best_idx = lval, lidx
        else:
            take = lval < best_val
            best_idx = jnp.where(take, lidx, best_idx)
            best_val = jnp.where(take, lval, best_val)

    idx_ref[0, 0, :] = best_idx
    part = jnp.sum(best_val).reshape(1, 1)
    acc = jnp.where(tb == 0, part, loss_ref[...] + part)
    loss_ref[...] = jnp.where(tb == NT - 1, acc * LOSS_SCALE, acc)


def _vq_tc(x, weight):
    return pl.pallas_call(
        _vq_tc_body,
        grid=(NT,),
        in_specs=[
            pl.BlockSpec((TM, DIM), lambda tb: (tb, 0)),
            pl.BlockSpec((NUM_CODES, DIM), lambda tb: (0, 0)),
        ],
        out_specs=[
            pl.BlockSpec((1, 1, TM), lambda tb: (tb, 0, 0)),
            pl.BlockSpec((1, 1), lambda tb: (0, 0)),
        ],
        out_shape=[
            jax.ShapeDtypeStruct((NT, 1, TM), jnp.int32),
            jax.ShapeDtypeStruct((1, 1), jnp.float32),
        ],
        scratch_shapes=[
            pltpu.VMEM((NUM_CODES, DIM), jnp.bfloat16),
            pltpu.VMEM((NUM_CODES, 1), jnp.float32),
        ],
    )(x, weight)


def _sc_gather(weight, idx):
    n = idx.shape[0]
    idx2d = idx.reshape(1, n)
    window = 128
    mesh = plsc.VectorSubcoreMesh(core_axis_name="core",
                                  subcore_axis_name="subcore")

    @pl.kernel(out_type=jax.ShapeDtypeStruct((n, DIM), weight.dtype),
               mesh=mesh)
    def gather_kernel(w_hbm, i_hbm, o_hbm):
        def body(i_vmem, o_vmem):
            pltpu.sync_copy(w_hbm.at[i_vmem.at[0]], o_vmem)

        pltpu.emit_pipeline(
            body,
            grid=(n // window,),
            in_specs=[pl.BlockSpec((1, window), lambda i: (0, i))],
            out_specs=[pl.BlockSpec((window, DIM), lambda i: (i, 0))],
            core_axis_name=("core", "subcore"),
            dimension_semantics=(pltpu.PARALLEL,),
        )(i_hbm, o_hbm)

    return gather_kernel(weight, idx2d)


def kernel(inputs, weight):
    b, c, h, w = inputs.shape
    x = jnp.transpose(inputs, (0, 2, 3, 1)).reshape(-1, DIM)
    idx3, loss2 = _vq_tc(x, weight)
    idx = idx3.reshape(-1)
    q = _sc_gather(weight, idx)
    quantized = jnp.transpose(q.reshape(b, h, w, DIM), (0, 3, 1, 2))
    return quantized, loss2.reshape(()), idx.reshape(b, h, w)


# speedup vs baseline: 1.1701x; 1.1701x over previous
import jax
import jax.numpy as jnp
from jax.experimental import pallas as pl
from jax.experimental.pallas import tpu as pltpu
from jax.experimental.pallas import tpu_sc as plsc

NUM_CODES = 8192
DIM = 256
TM = 1024
TN = 1024
NT = 8192 // TM
NC = NUM_CODES // TN
LOSS_SCALE = 0.25 / 2097152.0


def _row_sumsq(a):
    aa = a * a
    return jnp.sum(aa[:, :128] + aa[:, 128:], axis=1)


def _argmin_cols(d, base):
    rows = d.shape[0]
    vals = d
    idxs = jax.lax.broadcasted_iota(jnp.int32, d.shape, 0) + base
    while rows > 1:
        half = rows // 2
        a_v, b_v = vals[:half], vals[half:]
        a_i, b_i = idxs[:half], idxs[half:]
        take_b = (b_v < a_v) | ((b_v == a_v) & (b_i < a_i))
        vals = jnp.where(take_b, b_v, a_v)
        idxs = jnp.where(take_b, b_i, a_i)
        rows = half
    return vals[0], idxs[0]


def _vq_tc_body(x_ref, w_ref, idx_ref, loss_ref):
    tb = pl.program_id(0)
    x = x_ref[...]
    sx = _row_sumsq(x)
    x16 = x.astype(jnp.bfloat16)

    best_val = None
    best_idx = None
    for ci in range(NC):
        w = w_ref[ci * TN:(ci + 1) * TN, :]
        sw = _row_sumsq(w)
        mm = jax.lax.dot_general(
            w.astype(jnp.bfloat16), x16,
            (((1,), (1,)), ((), ())),
            preferred_element_type=jnp.float32)
        d = (sw[:, None] + sx[None, :]) - 2.0 * mm
        lval, lidx = _argmin_cols(d, ci * TN)
        if ci == 0:
            best_val, best_idx = lval, lidx
        else:
            take = lval < best_val
            best_idx = jnp.where(take, lidx, best_idx)
            best_val = jnp.where(take, lval, best_val)

    idx_ref[0, 0, :] = best_idx
    part = jnp.sum(best_val).reshape(1, 1)
    acc = jnp.where(tb == 0, part, loss_ref[...] + part)
    loss_ref[...] = jnp.where(tb == NT - 1, acc * LOSS_SCALE, acc)


def _vq_tc(x, weight):
    return pl.pallas_call(
        _vq_tc_body,
        grid=(NT,),
        in_specs=[
            pl.BlockSpec((TM, DIM), lambda tb: (tb, 0)),
            pl.BlockSpec((NUM_CODES, DIM), lambda tb: (0, 0)),
        ],
        out_specs=[
            pl.BlockSpec((1, 1, TM), lambda tb: (tb, 0, 0)),
            pl.BlockSpec((1, 1), lambda tb: (0, 0)),
        ],
        out_shape=[
            jax.ShapeDtypeStruct((NT, 1, TM), jnp.int32),
            jax.ShapeDtypeStruct((1, 1), jnp.float32),
        ],
    )(x, weight)


def _sc_gather(weight, idx):
    n = idx.shape[0]
    idx2d = idx.reshape(1, n)
    window = 128
    mesh = plsc.VectorSubcoreMesh(core_axis_name="core",
                                  subcore_axis_name="subcore")

    @pl.kernel(out_type=jax.ShapeDtypeStruct((n, DIM), weight.dtype),
               mesh=mesh)
    def gather_kernel(w_hbm, i_hbm, o_hbm):
        def body(i_vmem, o_vmem):
            pltpu.sync_copy(w_hbm.at[i_vmem.at[0]], o_vmem)

        pltpu.emit_pipeline(
            body,
            grid=(n // window,),
            in_specs=[pl.BlockSpec((1, window), lambda i: (0, i))],
            out_specs=[pl.BlockSpec((window, DIM), lambda i: (i, 0))],
            core_axis_name=("core", "subcore"),
            dimension_semantics=(pltpu.PARALLEL,),
        )(i_hbm, o_hbm)

    return gather_kernel(weight, idx2d)


def kernel(inputs, weight):
    b, c, h, w = inputs.shape
    x = jnp.transpose(inputs, (0, 2, 3, 1)).reshape(-1, DIM)
    idx3, loss2 = _vq_tc(x, weight)
    idx = idx3.reshape(-1)
    q = _sc_gather(weight, idx)
    quantized = jnp.transpose(q.reshape(b, h, w, DIM), (0, 3, 1, 2))
    return quantized, loss2.reshape(()), idx.reshape(b, h, w)
